# baseline (speedup 1.0000x reference)
import functools

import jax
import jax.numpy as jnp
from jax import lax
from jax.experimental import pallas as pl
from jax.experimental.pallas import tpu as pltpu
from jax.experimental.pallas import tpu_sc as plsc

NUM_WORKERS = 32


def _sc_gather(table, idx, chunk, tc_tiling=True):
    E = idx.shape[0]
    D = table.shape[1]
    per_w = E // NUM_WORKERS
    assert per_w * NUM_WORKERS == E and per_w % chunk == 0 and chunk % 8 == 0
    n_chunks = per_w // chunk
    mesh = plsc.VectorSubcoreMesh(core_axis_name="c", subcore_axis_name="s")

    @functools.partial(
        pl.kernel,
        out_type=jax.ShapeDtypeStruct((E, D), table.dtype),
        mesh=mesh,
        scratch_types=[
            pltpu.VMEM((per_w,), jnp.int32),
            pltpu.VMEM((chunk, D), table.dtype),
            pltpu.VMEM((chunk, D), table.dtype),
            pltpu.SemaphoreType.DMA,
            pltpu.SemaphoreType.DMA,
            pltpu.SemaphoreType.DMA,
            pltpu.SemaphoreType.DMA,
        ],
        compiler_params=pltpu.CompilerParams(use_tc_tiling_on_sc=tc_tiling),
    )
    def gather_kernel(table_hbm, idx_hbm, out_hbm, idx_v, buf0, buf1,
                      gs0, gs1, ss0, ss1):
        wid = lax.axis_index("s") * 2 + lax.axis_index("c")
        base = wid * per_w
        pltpu.sync_copy(idx_hbm.at[pl.ds(base, per_w)], idx_v)
        bufs, gsems, ssems = (buf0, buf1), (gs0, gs1), (ss0, ss1)
        stores = [None, None]
        for c in range(n_chunks):
            p = c % 2
            if stores[p] is not None:
                stores[p].wait()
            g = pltpu.async_copy(
                table_hbm.at[idx_v.at[pl.ds(c * chunk, chunk)]], bufs[p], gsems[p])
            g.wait()
            stores[p] = pltpu.async_copy(
                bufs[p], out_hbm.at[pl.ds(base + c * chunk, chunk)], ssems[p])
        for s in stores:
            if s is not None:
                s.wait()

    return gather_kernel(table, idx)


def _edge_logits(nbr, coords_pad, W1p, b1r, W2r, K, block):
    E = nbr.shape[0]
    N = coords_pad.shape[0]
    nblk = N // block

    def body(nbr_ref, cen_ref, w1_ref, b1_ref, w2_ref, out_ref):
        rel = nbr_ref[...] - cen_ref[...]
        h = (
            rel[:, 0:1] * w1_ref[0:1, :]
            + rel[:, 1:2] * w1_ref[1:2, :]
            + rel[:, 2:3] * w1_ref[2:3, :]
            + b1_ref[...]
        )
        h = 0.5 * h * (1.0 + lax.erf(h * 0.7071067811865476))
        out_ref[...] = jnp.sum(h * w2_ref[...], axis=1, keepdims=True)

    return pl.pallas_call(
        body,
        grid=(K, nblk),
        in_specs=[
            pl.BlockSpec((block, 16), lambda k, i: (k * nblk + i, 0)),
            pl.BlockSpec((block, 16), lambda k, i: (i, 0)),
            pl.BlockSpec((8, 64), lambda k, i: (0, 0)),
            pl.BlockSpec((1, 64), lambda k, i: (0, 0)),
            pl.BlockSpec((1, 64), lambda k, i: (0, 0)),
        ],
        out_specs=pl.BlockSpec((block, 1), lambda k, i: (k * nblk + i, 0)),
        out_shape=jax.ShapeDtypeStruct((E, 1), jnp.float32),
    )(nbr, coords_pad, W1p, b1r, W2r)


def _softmax_k(logits, block):
    N, K = logits.shape

    def body(x_ref, o_ref):
        x = x_ref[...]
        m = jnp.max(x, axis=1, keepdims=True)
        e = jnp.exp(x - m)
        o_ref[...] = e / jnp.sum(e, axis=1, keepdims=True)

    return pl.pallas_call(
        body,
        grid=(N // block,),
        in_specs=[pl.BlockSpec((block, K), lambda i: (i, 0))],
        out_specs=pl.BlockSpec((block, K), lambda i: (i, 0)),
        out_shape=jax.ShapeDtypeStruct((N, K), jnp.float32),
    )(logits)


def _step(g, ew, feat, Wf, Wa, bgr, gammar, betar, block, row_off):
    K, Nh, D = g.shape
    N = feat.shape[0]

    def body(g_ref, w_ref, f_ref, wf_ref, wa_ref, bg_ref, gm_ref, bt_ref, o_ref):
        w_blk = w_ref[...]
        agg = g_ref[0] * w_blk[:, 0:1]
        for k in range(1, K):
            agg = agg + g_ref[k] * w_blk[:, k : k + 1]
        f = f_ref[...]
        fh = f.astype(jnp.bfloat16)
        fl = (f - fh.astype(jnp.float32)).astype(jnp.bfloat16)
        ah = agg.astype(jnp.bfloat16)
        al = (agg - ah.astype(jnp.float32)).astype(jnp.bfloat16)
        wf = wf_ref[...]
        wa = wa_ref[...]
        z = (
            jnp.dot(fh, wf, preferred_element_type=jnp.float32)
            + jnp.dot(fl, wf, preferred_element_type=jnp.float32)
            + jnp.dot(ah, wa, preferred_element_type=jnp.float32)
            + jnp.dot(al, wa, preferred_element_type=jnp.float32)
            + bg_ref[...]
        )
        gate = jax.nn.sigmoid(z)
        upd = f + gate * (agg - f)
        mean = jnp.mean(upd, axis=1, keepdims=True)
        cen = upd - mean
        var = jnp.mean(cen * cen, axis=1, keepdims=True)
        o_ref[...] = cen * lax.rsqrt(var + 1e-5) * gm_ref[...] + bt_ref[...]

    return pl.pallas_call(
        body,
        grid=(Nh // block,),
        in_specs=[
            pl.BlockSpec((K, block, D), lambda i: (0, i, 0)),
            pl.BlockSpec((block, K), lambda i: (row_off + i, 0)),
            pl.BlockSpec((block, D), lambda i: (row_off + i, 0)),
            pl.BlockSpec((D, D), lambda i: (0, 0)),
            pl.BlockSpec((D, D), lambda i: (0, 0)),
            pl.BlockSpec((1, D), lambda i: (0, 0)),
            pl.BlockSpec((1, D), lambda i: (0, 0)),
            pl.BlockSpec((1, D), lambda i: (0, 0)),
        ],
        out_specs=pl.BlockSpec((block, D), lambda i: (i, 0)),
        out_shape=jax.ShapeDtypeStruct((Nh, D), jnp.float32),
    )(g, ew, feat, Wf, Wa, bgr, gammar, betar)


def kernel(feat, coords, knn_idx, W1, b1, W2, b2, Wg, bg, gamma, beta):
    N, D = feat.shape
    K = knn_idx.shape[1]
    E = N * K
    num_steps = Wg.shape[0]

    coords_pad = jnp.zeros((N, 16), jnp.float32).at[:, :3].set(coords)
    knn_i32 = knn_idx.astype(jnp.int32)
    feat_idx = knn_i32.T.reshape(E)
    Nh = N // 2
    idx0 = knn_i32[:Nh].T.reshape(K * Nh)
    idx1 = knn_i32[Nh:].T.reshape(K * Nh)

    W1p = jnp.zeros((8, 64), jnp.float32).at[:3].set(W1)
    b1r = b1.reshape(1, 64)
    W2r = W2.reshape(1, 64)
    Wf = Wg[:, :D, :].astype(jnp.bfloat16)
    Wa = Wg[:, D:, :].astype(jnp.bfloat16)
    bgr = bg.reshape(num_steps, 1, D)
    gammar = gamma.reshape(num_steps, 1, D)
    betar = beta.reshape(num_steps, 1, D)

    nbr = _sc_gather(coords_pad, feat_idx, chunk=2000, tc_tiling=False)
    logits = _edge_logits(nbr, coords_pad, W1p, b1r, W2r, K, block=2000)
    ew = _softmax_k(logits.reshape(K, N).T, block=2000)

    blk = 1000
    for t in range(num_steps):
        g0 = _sc_gather(feat, idx0, chunk=200).reshape(K, Nh, D)
        g1 = _sc_gather(feat, idx1, chunk=200).reshape(K, Nh, D)
        f0 = _step(g0, ew, feat, Wf[t], Wa[t], bgr[t], gammar[t], betar[t],
                   block=blk, row_off=0)
        f1 = _step(g1, ew, feat, Wf[t], Wa[t], bgr[t], gammar[t], betar[t],
                   block=blk, row_off=Nh // blk)
        feat = jnp.concatenate([f0, f1], axis=0)
    return feat


# speedup vs baseline: 1.0188x; 1.0188x over previous
import functools

import jax
import jax.numpy as jnp
from jax import lax
from jax.experimental import pallas as pl
from jax.experimental.pallas import tpu as pltpu
from jax.experimental.pallas import tpu_sc as plsc

NUM_WORKERS = 32


def _sc_gather(table, idx, chunk, tc_tiling=True):
    E = idx.shape[0]
    D = table.shape[1]
    per_w = E // NUM_WORKERS
    assert per_w * NUM_WORKERS == E and per_w % chunk == 0 and chunk % 8 == 0
    n_chunks = per_w // chunk
    mesh = plsc.VectorSubcoreMesh(core_axis_name="c", subcore_axis_name="s")

    @functools.partial(
        pl.kernel,
        out_type=jax.ShapeDtypeStruct((E, D), table.dtype),
        mesh=mesh,
        scratch_types=[
            pltpu.VMEM((per_w,), jnp.int32),
            pltpu.VMEM((chunk, D), table.dtype),
            pltpu.VMEM((chunk, D), table.dtype),
            pltpu.SemaphoreType.DMA,
            pltpu.SemaphoreType.DMA,
            pltpu.SemaphoreType.DMA,
            pltpu.SemaphoreType.DMA,
        ],
        compiler_params=pltpu.CompilerParams(use_tc_tiling_on_sc=tc_tiling),
    )
    def gather_kernel(table_hbm, idx_hbm, out_hbm, idx_v, buf0, buf1,
                      gs0, gs1, ss0, ss1):
        wid = lax.axis_index("s") * 2 + lax.axis_index("c")
        base = wid * per_w
        pltpu.sync_copy(idx_hbm.at[pl.ds(base, per_w)], idx_v)
        bufs, gsems, ssems = (buf0, buf1), (gs0, gs1), (ss0, ss1)
        stores = [None, None]
        for c in range(n_chunks):
            p = c % 2
            if stores[p] is not None:
                stores[p].wait()
            g = pltpu.async_copy(
                table_hbm.at[idx_v.at[pl.ds(c * chunk, chunk)]], bufs[p], gsems[p])
            g.wait()
            stores[p] = pltpu.async_copy(
                bufs[p], out_hbm.at[pl.ds(base + c * chunk, chunk)], ssems[p])
        for s in stores:
            if s is not None:
                s.wait()

    return gather_kernel(table, idx)


def _edge_logits(nbr, coords_pad, W1p, b1r, W2r, K, block):
    E = nbr.shape[0]
    N = coords_pad.shape[0]
    nblk = N // block

    def body(nbr_ref, cen_ref, w1_ref, b1_ref, w2_ref, out_ref):
        nb = nbr_ref[...]
        ce = cen_ref[...]
        h = (
            (nb[:, 0:1] - ce[:, 0:1]) * w1_ref[0:1, :]
            + (nb[:, 1:2] - ce[:, 1:2]) * w1_ref[1:2, :]
            + (nb[:, 2:3] - ce[:, 2:3]) * w1_ref[2:3, :]
            + b1_ref[...]
        )
        h = 0.5 * h * (1.0 + lax.erf(h * 0.7071067811865476))
        logit = jnp.dot(h.astype(jnp.bfloat16), w2_ref[...],
                        preferred_element_type=jnp.float32)
        out_ref[...] = logit[:, 0:1]

    return pl.pallas_call(
        body,
        grid=(K, nblk),
        in_specs=[
            pl.BlockSpec((block, 128), lambda k, i: (k * nblk + i, 0)),
            pl.BlockSpec((block, 128), lambda k, i: (i, 0)),
            pl.BlockSpec((8, 64), lambda k, i: (0, 0)),
            pl.BlockSpec((1, 64), lambda k, i: (0, 0)),
            pl.BlockSpec((64, 8), lambda k, i: (0, 0)),
        ],
        out_specs=pl.BlockSpec((block, 1), lambda k, i: (k * nblk + i, 0)),
        out_shape=jax.ShapeDtypeStruct((E, 1), jnp.float32),
    )(nbr, coords_pad, W1p, b1r, W2r)


def _softmax_k(logits, block):
    N, K = logits.shape

    def body(x_ref, o_ref):
        x = x_ref[...]
        m = jnp.max(x, axis=1, keepdims=True)
        e = jnp.exp(x - m)
        o_ref[...] = e / jnp.sum(e, axis=1, keepdims=True)

    return pl.pallas_call(
        body,
        grid=(N // block,),
        in_specs=[pl.BlockSpec((block, K), lambda i: (i, 0))],
        out_specs=pl.BlockSpec((block, K), lambda i: (i, 0)),
        out_shape=jax.ShapeDtypeStruct((N, K), jnp.float32),
    )(logits)


def _step(g, ew, feat, Wf, Wa, bgr, gammar, betar, block, row_off):
    K, Nh, D = g.shape
    N = feat.shape[0]

    def body(g_ref, w_ref, f_ref, wf_ref, wa_ref, bg_ref, gm_ref, bt_ref, o_ref):
        w_blk = w_ref[...]
        agg = g_ref[0] * w_blk[:, 0:1]
        for k in range(1, K):
            agg = agg + g_ref[k] * w_blk[:, k : k + 1]
        f = f_ref[...]
        fh = f.astype(jnp.bfloat16)
        fl = (f - fh.astype(jnp.float32)).astype(jnp.bfloat16)
        ah = agg.astype(jnp.bfloat16)
        al = (agg - ah.astype(jnp.float32)).astype(jnp.bfloat16)
        wf = wf_ref[...]
        wa = wa_ref[...]
        z = (
            jnp.dot(fh, wf, preferred_element_type=jnp.float32)
            + jnp.dot(fl, wf, preferred_element_type=jnp.float32)
            + jnp.dot(ah, wa, preferred_element_type=jnp.float32)
            + jnp.dot(al, wa, preferred_element_type=jnp.float32)
            + bg_ref[...]
        )
        gate = jax.nn.sigmoid(z)
        upd = f + gate * (agg - f)
        mean = jnp.mean(upd, axis=1, keepdims=True)
        cen = upd - mean
        var = jnp.mean(cen * cen, axis=1, keepdims=True)
        o_ref[...] = cen * lax.rsqrt(var + 1e-5) * gm_ref[...] + bt_ref[...]

    return pl.pallas_call(
        body,
        grid=(Nh // block,),
        in_specs=[
            pl.BlockSpec((K, block, D), lambda i: (0, i, 0)),
            pl.BlockSpec((block, K), lambda i: (row_off + i, 0)),
            pl.BlockSpec((block, D), lambda i: (row_off + i, 0)),
            pl.BlockSpec((D, D), lambda i: (0, 0)),
            pl.BlockSpec((D, D), lambda i: (0, 0)),
            pl.BlockSpec((1, D), lambda i: (0, 0)),
            pl.BlockSpec((1, D), lambda i: (0, 0)),
            pl.BlockSpec((1, D), lambda i: (0, 0)),
        ],
        out_specs=pl.BlockSpec((block, D), lambda i: (i, 0)),
        out_shape=jax.ShapeDtypeStruct((Nh, D), jnp.float32),
    )(g, ew, feat, Wf, Wa, bgr, gammar, betar)


def kernel(feat, coords, knn_idx, W1, b1, W2, b2, Wg, bg, gamma, beta):
    N, D = feat.shape
    K = knn_idx.shape[1]
    E = N * K
    num_steps = Wg.shape[0]

    coords_pad = jnp.zeros((N, 128), jnp.float32).at[:, :3].set(coords)
    feat_idx = knn_idx.astype(jnp.int32).T.reshape(E)

    W1p = jnp.zeros((8, 64), jnp.float32).at[:3].set(W1)
    b1r = b1.reshape(1, 64)
    W2r = jnp.zeros((64, 8), jnp.float32).at[:, 0:1].set(W2).astype(jnp.bfloat16)
    Wf = Wg[:, :D, :].astype(jnp.bfloat16)
    Wa = Wg[:, D:, :].astype(jnp.bfloat16)
    bgr = bg.reshape(num_steps, 1, D)
    gammar = gamma.reshape(num_steps, 1, D)
    betar = beta.reshape(num_steps, 1, D)

    nbr = _sc_gather(coords_pad, feat_idx, chunk=400)
    logits = _edge_logits(nbr, coords_pad, W1p, b1r, W2r, K, block=2000)
    ew = _softmax_k(logits.reshape(K, N).T, block=2000)

    for t in range(num_steps):
        g = _sc_gather(feat, feat_idx, chunk=400).reshape(K, N, D)
        feat = _step(g, ew, feat, Wf[t], Wa[t], bgr[t], gammar[t], betar[t],
                     block=400, row_off=0)
    return feat


# speedup vs baseline: 1.1701x; 1.1485x over previous
import functools

import jax
import jax.numpy as jnp
from jax import lax
from jax.experimental import pallas as pl
from jax.experimental.pallas import tpu as pltpu
from jax.experimental.pallas import tpu_sc as plsc

NUM_WORKERS = 32


def _sc_gather(table, idx, chunk, tc_tiling=True):
    E = idx.shape[0]
    D = table.shape[1]
    per_w = E // NUM_WORKERS
    assert per_w * NUM_WORKERS == E and per_w % chunk == 0 and chunk % 8 == 0
    n_chunks = per_w // chunk
    mesh = plsc.VectorSubcoreMesh(core_axis_name="c", subcore_axis_name="s")

    @functools.partial(
        pl.kernel,
        out_type=jax.ShapeDtypeStruct((E, D), table.dtype),
        mesh=mesh,
        scratch_types=[
            pltpu.VMEM((per_w,), jnp.int32),
            pltpu.VMEM((chunk, D), table.dtype),
            pltpu.VMEM((chunk, D), table.dtype),
            pltpu.SemaphoreType.DMA,
            pltpu.SemaphoreType.DMA,
            pltpu.SemaphoreType.DMA,
            pltpu.SemaphoreType.DMA,
        ],
        compiler_params=pltpu.CompilerParams(use_tc_tiling_on_sc=tc_tiling),
    )
    def gather_kernel(table_hbm, idx_hbm, out_hbm, idx_v, buf0, buf1,
                      gs0, gs1, ss0, ss1):
        wid = lax.axis_index("s") * 2 + lax.axis_index("c")
        base = wid * per_w
        pltpu.sync_copy(idx_hbm.at[pl.ds(base, per_w)], idx_v)
        bufs, gsems, ssems = (buf0, buf1), (gs0, gs1), (ss0, ss1)
        stores = [None, None]
        for c in range(n_chunks):
            p = c % 2
            if stores[p] is not None:
                stores[p].wait()
            g = pltpu.async_copy(
                table_hbm.at[idx_v.at[pl.ds(c * chunk, chunk)]], bufs[p], gsems[p])
            g.wait()
            stores[p] = pltpu.async_copy(
                bufs[p], out_hbm.at[pl.ds(base + c * chunk, chunk)], ssems[p])
        for s in stores:
            if s is not None:
                s.wait()

    return gather_kernel(table, idx)


def _edge_softmax(nbr3, coords_pad, W1p, b1r, W2r, block):
    K, N, _ = nbr3.shape

    def body(nbr_ref, cen_ref, w1_ref, b1_ref, w2_ref, o_ref):
        ce = cen_ref[...]
        cols = []
        for k in range(K):
            nb = nbr_ref[k]
            h = (
                (nb[:, 0:1] - ce[:, 0:1]) * w1_ref[0:1, :]
                + (nb[:, 1:2] - ce[:, 1:2]) * w1_ref[1:2, :]
                + (nb[:, 2:3] - ce[:, 2:3]) * w1_ref[2:3, :]
                + b1_ref[...]
            )
            h = 0.5 * h * (1.0 + lax.erf(h * 0.7071067811865476))
            logit = jnp.dot(h.astype(jnp.bfloat16), w2_ref[...],
                            preferred_element_type=jnp.float32)
            cols.append(logit[:, 0:1])
        x = jnp.concatenate(cols, axis=1)
        m = jnp.max(x, axis=1, keepdims=True)
        e = jnp.exp(x - m)
        o_ref[...] = e / jnp.sum(e, axis=1, keepdims=True)

    return pl.pallas_call(
        body,
        grid=(N // block,),
        in_specs=[
            pl.BlockSpec((K, block, 128), lambda i: (0, i, 0)),
            pl.BlockSpec((block, 128), lambda i: (i, 0)),
            pl.BlockSpec((8, 64), lambda i: (0, 0)),
            pl.BlockSpec((1, 64), lambda i: (0, 0)),
            pl.BlockSpec((64, 8), lambda i: (0, 0)),
        ],
        out_specs=pl.BlockSpec((block, K), lambda i: (i, 0)),
        out_shape=jax.ShapeDtypeStruct((N, K), jnp.float32),
    )(nbr3, coords_pad, W1p, b1r, W2r)


def _step(g, ew, feat, Wf, Wa, bgr, gammar, betar, block, row_off):
    K, Nh, D = g.shape
    N = feat.shape[0]

    def body(g_ref, w_ref, f_ref, wf_ref, wa_ref, bg_ref, gm_ref, bt_ref, o_ref):
        w_blk = w_ref[...]
        agg = g_ref[0] * w_blk[:, 0:1]
        for k in range(1, K):
            agg = agg + g_ref[k] * w_blk[:, k : k + 1]
        f = f_ref[...]
        fh = f.astype(jnp.bfloat16)
        fl = (f - fh.astype(jnp.float32)).astype(jnp.bfloat16)
        ah = agg.astype(jnp.bfloat16)
        al = (agg - ah.astype(jnp.float32)).astype(jnp.bfloat16)
        wf = wf_ref[...]
        wa = wa_ref[...]
        z = (
            jnp.dot(fh, wf, preferred_element_type=jnp.float32)
            + jnp.dot(fl, wf, preferred_element_type=jnp.float32)
            + jnp.dot(ah, wa, preferred_element_type=jnp.float32)
            + jnp.dot(al, wa, preferred_element_type=jnp.float32)
            + bg_ref[...]
        )
        gate = jax.nn.sigmoid(z)
        upd = f + gate * (agg - f)
        mean = jnp.mean(upd, axis=1, keepdims=True)
        cen = upd - mean
        var = jnp.mean(cen * cen, axis=1, keepdims=True)
        o_ref[...] = cen * lax.rsqrt(var + 1e-5) * gm_ref[...] + bt_ref[...]

    return pl.pallas_call(
        body,
        grid=(Nh // block,),
        in_specs=[
            pl.BlockSpec((K, block, D), lambda i: (0, i, 0)),
            pl.BlockSpec((block, K), lambda i: (row_off + i, 0)),
            pl.BlockSpec((block, D), lambda i: (row_off + i, 0)),
            pl.BlockSpec((D, D), lambda i: (0, 0)),
            pl.BlockSpec((D, D), lambda i: (0, 0)),
            pl.BlockSpec((1, D), lambda i: (0, 0)),
            pl.BlockSpec((1, D), lambda i: (0, 0)),
            pl.BlockSpec((1, D), lambda i: (0, 0)),
        ],
        out_specs=pl.BlockSpec((block, D), lambda i: (i, 0)),
        out_shape=jax.ShapeDtypeStruct((Nh, D), jnp.float32),
    )(g, ew, feat, Wf, Wa, bgr, gammar, betar)


def kernel(feat, coords, knn_idx, W1, b1, W2, b2, Wg, bg, gamma, beta):
    N, D = feat.shape
    K = knn_idx.shape[1]
    E = N * K
    num_steps = Wg.shape[0]

    coords_pad = jnp.zeros((N, 128), jnp.float32).at[:, :3].set(coords)
    feat_idx = knn_idx.astype(jnp.int32).T.reshape(E)

    W1p = jnp.zeros((8, 64), jnp.float32).at[:3].set(W1)
    b1r = b1.reshape(1, 64)
    W2r = jnp.zeros((64, 8), jnp.float32).at[:, 0:1].set(W2).astype(jnp.bfloat16)
    Wf = Wg[:, :D, :].astype(jnp.bfloat16)
    Wa = Wg[:, D:, :].astype(jnp.bfloat16)
    bgr = bg.reshape(num_steps, 1, D)
    gammar = gamma.reshape(num_steps, 1, D)
    betar = beta.reshape(num_steps, 1, D)

    nbr = _sc_gather(coords_pad, feat_idx, chunk=400)
    ew = _edge_softmax(nbr.reshape(K, N, 128), coords_pad,
                       W1p, b1r, W2r, block=400)

    for t in range(num_steps):
        g = _sc_gather(feat, feat_idx, chunk=400).reshape(K, N, D)
        feat = _step(g, ew, feat, Wf[t], Wa[t], bgr[t], gammar[t], betar[t],
                     block=400, row_off=0)
    return feat


# speedup vs baseline: 1.1707x; 1.0005x over previous
import functools

import jax
import jax.numpy as jnp
from jax import lax
from jax.experimental import pallas as pl
from jax.experimental.pallas import tpu as pltpu
from jax.experimental.pallas import tpu_sc as plsc

NUM_WORKERS = 32


def _sc_gather(table, idx, chunk):
    E = idx.shape[0]
    D = table.shape[1]
    per_w = E // NUM_WORKERS
    assert per_w * NUM_WORKERS == E and per_w % chunk == 0 and chunk % 8 == 0
    n_chunks = per_w // chunk
    mesh = plsc.VectorSubcoreMesh(core_axis_name="c", subcore_axis_name="s")

    @functools.partial(
        pl.kernel,
        out_type=jax.ShapeDtypeStruct((E, D), table.dtype),
        mesh=mesh,
        scratch_types=[
            pltpu.VMEM((per_w,), jnp.int32),
            pltpu.VMEM((chunk, D), table.dtype),
            pltpu.VMEM((chunk, D), table.dtype),
            pltpu.SemaphoreType.DMA,
            pltpu.SemaphoreType.DMA,
            pltpu.SemaphoreType.DMA,
            pltpu.SemaphoreType.DMA,
        ],
    )
    def gather_kernel(table_hbm, idx_hbm, out_hbm, idx_v, buf0, buf1,
                      gs0, gs1, ss0, ss1):
        wid = lax.axis_index("s") * 2 + lax.axis_index("c")
        base = wid * per_w
        pltpu.sync_copy(idx_hbm.at[pl.ds(base, per_w)], idx_v)
        bufs, gsems, ssems = (buf0, buf1), (gs0, gs1), (ss0, ss1)
        stores = [None, None]
        for c in range(n_chunks):
            p = c % 2
            if stores[p] is not None:
                stores[p].wait()
            g = pltpu.async_copy(
                table_hbm.at[idx_v.at[pl.ds(c * chunk, chunk)]], bufs[p], gsems[p])
            g.wait()
            stores[p] = pltpu.async_copy(
                bufs[p], out_hbm.at[pl.ds(base + c * chunk, chunk)], ssems[p])
        for s in stores:
            if s is not None:
                s.wait()

    return gather_kernel(table, idx)


def _edge_softmax(nbr3, coords_pad, W1p, b1r, W2r, block):
    K, N, _ = nbr3.shape

    def body(nbr_ref, cen_ref, w1_ref, b1_ref, w2_ref, o_ref):
        ce = cen_ref[...]
        cols = []
        for k in range(K):
            nb = nbr_ref[k]
            h = (
                (nb[:, 0:1] - ce[:, 0:1]) * w1_ref[0:1, :]
                + (nb[:, 1:2] - ce[:, 1:2]) * w1_ref[1:2, :]
                + (nb[:, 2:3] - ce[:, 2:3]) * w1_ref[2:3, :]
                + b1_ref[...]
            )
            h = 0.5 * h * (1.0 + lax.erf(h * 0.7071067811865476))
            logit = jnp.dot(h.astype(jnp.bfloat16), w2_ref[...],
                            preferred_element_type=jnp.float32)
            cols.append(logit[:, 0:1])
        x = jnp.concatenate(cols, axis=1)
        m = jnp.max(x, axis=1, keepdims=True)
        e = jnp.exp(x - m)
        o_ref[...] = e / jnp.sum(e, axis=1, keepdims=True)

    return pl.pallas_call(
        body,
        grid=(N // block,),
        in_specs=[
            pl.BlockSpec((K, block, 128), lambda i: (0, i, 0)),
            pl.BlockSpec((block, 128), lambda i: (i, 0)),
            pl.BlockSpec((8, 64), lambda i: (0, 0)),
            pl.BlockSpec((1, 64), lambda i: (0, 0)),
            pl.BlockSpec((64, 8), lambda i: (0, 0)),
        ],
        out_specs=pl.BlockSpec((block, K), lambda i: (i, 0)),
        out_shape=jax.ShapeDtypeStruct((N, K), jnp.float32),
    )(nbr3, coords_pad, W1p, b1r, W2r)


def _step(g, ew, feat, Wf, Wa, bgr, gammar, betar, block, row_off):
    K, Nh, D = g.shape
    N = feat.shape[0]

    def body(g_ref, w_ref, f_ref, wf_ref, wa_ref, bg_ref, gm_ref, bt_ref, o_ref):
        w_blk = w_ref[...]
        agg = g_ref[0] * w_blk[:, 0:1]
        for k in range(1, K):
            agg = agg + g_ref[k] * w_blk[:, k : k + 1]
        f = f_ref[...]
        fh = f.astype(jnp.bfloat16)
        fl = (f - fh.astype(jnp.float32)).astype(jnp.bfloat16)
        ah = agg.astype(jnp.bfloat16)
        al = (agg - ah.astype(jnp.float32)).astype(jnp.bfloat16)
        wf = wf_ref[...]
        wa = wa_ref[...]
        z = (
            jnp.dot(fh, wf, preferred_element_type=jnp.float32)
            + jnp.dot(fl, wf, preferred_element_type=jnp.float32)
            + jnp.dot(ah, wa, preferred_element_type=jnp.float32)
            + jnp.dot(al, wa, preferred_element_type=jnp.float32)
            + bg_ref[...]
        )
        gate = jax.nn.sigmoid(z)
        upd = f + gate * (agg - f)
        mean = jnp.mean(upd, axis=1, keepdims=True)
        cen = upd - mean
        var = jnp.mean(cen * cen, axis=1, keepdims=True)
        o_ref[...] = cen * lax.rsqrt(var + 1e-5) * gm_ref[...] + bt_ref[...]

    return pl.pallas_call(
        body,
        grid=(Nh // block,),
        in_specs=[
            pl.BlockSpec((K, block, D), lambda i: (0, i, 0)),
            pl.BlockSpec((block, K), lambda i: (row_off + i, 0)),
            pl.BlockSpec((block, D), lambda i: (row_off + i, 0)),
            pl.BlockSpec((D, D), lambda i: (0, 0)),
            pl.BlockSpec((D, D), lambda i: (0, 0)),
            pl.BlockSpec((1, D), lambda i: (0, 0)),
            pl.BlockSpec((1, D), lambda i: (0, 0)),
            pl.BlockSpec((1, D), lambda i: (0, 0)),
        ],
        out_specs=pl.BlockSpec((block, D), lambda i: (i, 0)),
        out_shape=jax.ShapeDtypeStruct((Nh, D), jnp.float32),
    )(g, ew, feat, Wf, Wa, bgr, gammar, betar)


def kernel(feat, coords, knn_idx, W1, b1, W2, b2, Wg, bg, gamma, beta):
    N, D = feat.shape
    K = knn_idx.shape[1]
    E = N * K
    num_steps = Wg.shape[0]

    coords_pad = jnp.zeros((N, 128), jnp.float32).at[:, :3].set(coords)
    feat_idx = knn_idx.astype(jnp.int32).T.reshape(E)

    W1p = jnp.zeros((8, 64), jnp.float32).at[:3].set(W1)
    b1r = b1.reshape(1, 64)
    W2r = jnp.zeros((64, 8), jnp.float32).at[:, 0:1].set(W2).astype(jnp.bfloat16)
    Wf = Wg[:, :D, :].astype(jnp.bfloat16)
    Wa = Wg[:, D:, :].astype(jnp.bfloat16)
    bgr = bg.reshape(num_steps, 1, D)
    gammar = gamma.reshape(num_steps, 1, D)
    betar = beta.reshape(num_steps, 1, D)

    nbr = _sc_gather(coords_pad, feat_idx, chunk=400)
    ew = _edge_softmax(nbr.reshape(K, N, 128), coords_pad,
                       W1p, b1r, W2r, block=400)

    for t in range(num_steps):
        g = _sc_gather(feat, feat_idx, chunk=400).reshape(K, N, D)
        feat = _step(g, ew, feat, Wf[t], Wa[t], bgr[t], gammar[t], betar[t],
                     block=400, row_off=0)
    return feat
